# baseline (speedup 1.0000x reference)
import functools

import jax
import jax.numpy as jnp
from jax import lax
from jax.experimental import pallas as pl
from jax.experimental.pallas import tpu as pltpu
from jax.experimental.pallas import tpu_sc as plsc

N = 10000
E = 320000
D = 128
H = 128
O = 64

NC = 2
NS = 16
NW = NC * NS
ET = E // NW
CH = 80
NCH = ET // CH
RPT = 624
ZR = 16

_mesh = plsc.VectorSubcoreMesh(core_axis_name="c", subcore_axis_name="s")
_sc_params = pltpu.CompilerParams(needs_layout_passes=False)


def _worker_id():
    return lax.axis_index("s") * NC + lax.axis_index("c")


def _deg_body(dst_hbm, w_hbm, deg_hbm, dstv, wv, zbuf, sacc, ssem):
    cid = lax.axis_index("c")
    sid = lax.axis_index("s")
    wid = _worker_id()
    pltpu.sync_copy(dst_hbm.at[wid], dstv)
    pltpu.sync_copy(w_hbm.at[wid], wv)

    @pl.when(sid == 0)
    def _():
        def zb(i, _):
            zbuf[pl.ds(i * 16, 16)] = jnp.zeros((16,), jnp.float32)
            return 0
        lax.fori_loop(0, 2000 // 16, zb, 0)

        def zc(k, _):
            pltpu.sync_copy(zbuf, sacc.at[pl.ds(k * 2000, 2000)])
            return 0
        lax.fori_loop(0, N // 2000, zc, 0)

    plsc.subcore_barrier()

    def chunk(j, _):
        pltpu.sync_copy(wv.at[j], sacc.at[dstv.at[j]], add=True)
        return 0
    lax.fori_loop(0, NCH, chunk, 0)

    plsc.subcore_barrier()

    @pl.when(sid == 0)
    def _():
        def ro(k, _):
            pltpu.sync_copy(sacc.at[pl.ds(k * 2000, 2000)], zbuf)
            off = pl.multiple_of(cid * N + k * 2000, 8)
            pltpu.sync_copy(zbuf, deg_hbm.at[pl.ds(off, 2000)])
            return 0
        lax.fori_loop(0, N // 2000, ro, 0)


_deg_kernel = functools.partial(
    pl.kernel,
    out_type=jax.ShapeDtypeStruct((NC * N,), jnp.float32),
    mesh=_mesh,
    compiler_params=_sc_params,
    scratch_types=[
        pltpu.VMEM((NCH, CH), jnp.int32),
        pltpu.VMEM((NCH, CH), jnp.float32),
        pltpu.VMEM((2000,), jnp.float32),
        pltpu.VMEM_SHARED((N,), jnp.float32),
        pltpu.SemaphoreType.DMA,
    ],
)(_deg_body)


def _norm_body(src_hbm, dst_hbm, w_hbm, dis_hbm, norm_hbm,
               sv, dv, wv, disv, normv):
    wid = _worker_id()
    base = pl.multiple_of(wid * ET, 8)
    pltpu.sync_copy(src_hbm.at[pl.ds(base, ET)], sv)
    pltpu.sync_copy(dst_hbm.at[pl.ds(base, ET)], dv)
    pltpu.sync_copy(w_hbm.at[pl.ds(base, ET)], wv)
    pltpu.sync_copy(dis_hbm, disv)

    def grp(t, _):
        s16 = sv[pl.ds(t * 16, 16)]
        d16 = dv[pl.ds(t * 16, 16)]
        w16 = wv[pl.ds(t * 16, 16)]
        n16 = plsc.load_gather(disv, [s16]) * w16 * plsc.load_gather(disv, [d16])
        normv[pl.ds(t * 16, 16)] = n16
        return 0
    lax.fori_loop(0, ET // 16, grp, 0)

    pltpu.sync_copy(normv, norm_hbm.at[pl.ds(base, ET)])


_norm_kernel = functools.partial(
    pl.kernel,
    out_type=jax.ShapeDtypeStruct((E,), jnp.float32),
    mesh=_mesh,
    compiler_params=_sc_params,
    scratch_types=[
        pltpu.VMEM((ET,), jnp.int32),
        pltpu.VMEM((ET,), jnp.int32),
        pltpu.VMEM((ET,), jnp.float32),
        pltpu.VMEM((N,), jnp.float32),
        pltpu.VMEM((ET,), jnp.float32),
    ],
)(_norm_body)


def _scatter_body(width, mat_hbm, packed_hbm, out_hbm,
                  eb0, eb1, rb0, rb1, rb2, sacc,
                  gsem0, gsem1, ssem, esem0, esem1):
    cid = lax.axis_index("c")
    sid = lax.axis_index("s")
    wid = _worker_id()

    def zrows(i, _):
        for c in range(width // 16):
            rb2[i, pl.ds(c * 16, 16)] = jnp.zeros((16,), jnp.float32)
        return 0
    lax.fori_loop(0, ZR, zrows, 0)

    row0 = sid * RPT
    nz = jnp.where(sid == NS - 1, (N - (NS - 1) * RPT) // ZR, RPT // ZR)

    def zc(k, _):
        r0 = pl.multiple_of(row0 + k * ZR, 8)
        pltpu.sync_copy(rb2.at[pl.ds(0, ZR)], sacc.at[pl.ds(r0, ZR)])
        return 0
    lax.fori_loop(0, nz, zc, 0)

    plsc.subcore_barrier()

    lane_consts = [jnp.full((16,), r, jnp.int32) for r in range(16)]
    ebufs = ((eb0, esem0), (eb1, esem1))
    rbufs = ((rb0, gsem0), (rb1, gsem1))
    cbase = wid * NCH

    def efetch(j, b):
        eb, esem = ebufs[b]
        pltpu.async_copy(packed_hbm.at[cbase + j], eb, esem)

    def edrain(b):
        eb, esem = ebufs[b]
        pltpu.make_async_copy(packed_hbm.at[0], eb, esem).wait()

    def gather(b):
        eb = ebufs[b][0]
        rb, gsem = rbufs[b]
        pltpu.async_copy(mat_hbm.at[eb.at[0]], rb, gsem)

    def gwait(b):
        rb, gsem = rbufs[b]
        pltpu.make_async_copy(mat_hbm.at[pl.ds(0, CH)], rb, gsem).wait()

    def swait():
        pltpu.make_async_copy(mat_hbm.at[pl.ds(0, CH)], rb2, ssem).wait()

    def scale(b):
        eb = ebufs[b][0]
        rb = rbufs[b][0]

        def grp(g, _):
            n16 = plsc.bitcast(eb[2, pl.ds(g * 16, 16)], jnp.float32)
            for r0 in range(0, 16, 2):
                rows = [g * 16 + r0, g * 16 + r0 + 1]
                bcs = [jnp.take(n16, lane_consts[r0 + i]) for i in range(2)]
                vals = [[rb[rows[i], pl.ds(c * 16, 16)] for c in range(width // 16)]
                        for i in range(2)]
                for i in range(2):
                    for c in range(width // 16):
                        rb2[rows[i], pl.ds(c * 16, 16)] = vals[i][c] * bcs[i]
            return 0
        lax.fori_loop(0, CH // 16, grp, 0)

    def scat(b):
        eb = ebufs[b][0]
        pltpu.async_copy(rb2, sacc.at[eb.at[1]], ssem, add=True)

    efetch(0, 0)
    edrain(0)
    gather(0)

    def pair(k, _):
        @pl.when(k > 0)
        def _():
            swait()
        efetch(2 * k + 1, 1)
        edrain(1)
        gather(1)
        gwait(0)
        scale(0)
        scat(0)

        swait()
        efetch(2 * k + 2, 0)
        edrain(0)
        gather(0)
        gwait(1)
        scale(1)
        scat(1)
        return 0
    lax.fori_loop(0, (NCH - 1) // 2, pair, 0)

    swait()
    gwait(0)
    scale(0)
    pltpu.sync_copy(rb2, sacc.at[eb0.at[1]], add=True)

    plsc.subcore_barrier()

    def ro(k, _):
        r0 = pl.multiple_of(row0 + k * ZR, 8)
        pltpu.sync_copy(sacc.at[pl.ds(r0, ZR)], rb0.at[pl.ds(0, ZR)])
        pltpu.sync_copy(rb0.at[pl.ds(0, ZR)], out_hbm.at[cid, pl.ds(r0, ZR)])
        return 0
    lax.fori_loop(0, nz, ro, 0)


def _make_scatter_kernel(width):
    return functools.partial(
        pl.kernel,
        out_type=jax.ShapeDtypeStruct((NC, N, width), jnp.float32),
        mesh=_mesh,
        compiler_params=_sc_params,
        scratch_types=[
            pltpu.VMEM((3, CH), jnp.int32),
            pltpu.VMEM((3, CH), jnp.int32),
            pltpu.VMEM((CH, width), jnp.float32),
            pltpu.VMEM((CH, width), jnp.float32),
            pltpu.VMEM((CH, width), jnp.float32),
            pltpu.VMEM_SHARED((N, width), jnp.float32),
            pltpu.SemaphoreType.DMA,
            pltpu.SemaphoreType.DMA,
            pltpu.SemaphoreType.DMA,
            pltpu.SemaphoreType.DMA,
            pltpu.SemaphoreType.DMA,
        ],
    )(functools.partial(_scatter_body, width))


_scatter_h = _make_scatter_kernel(H)


def _tc_a_body(x_ref, w_ref, deg_ref, mat_ref, dis_ref, selfw_ref):
    deg = deg_ref[0:1, :] + deg_ref[1:2, :] + 1.0
    dis_ref[...] = lax.rsqrt(deg)
    selfw_ref[...] = 1.0 / deg
    mat_ref[...] = jnp.dot(x_ref[...], w_ref[...],
                           preferred_element_type=jnp.float32)


def _tc_a(x, w1, deg_part):
    return pl.pallas_call(
        _tc_a_body,
        out_shape=[
            jax.ShapeDtypeStruct((N, H), jnp.float32),
            jax.ShapeDtypeStruct((1, N), jnp.float32),
            jax.ShapeDtypeStruct((1, N), jnp.float32),
        ],
    )(x, w1, deg_part)


def _tc_b_body(p_ref, mat_ref, selfw_ref, b_ref, wn_ref, out_ref):
    h = p_ref[0] + p_ref[1] + selfw_ref[...] * mat_ref[...] + b_ref[...]
    h = jnp.maximum(h, 0.0)
    out_ref[...] = jnp.dot(h, wn_ref[...], preferred_element_type=jnp.float32)


def _tc_b(part, mat, selfw_col, b, w_next, width_out):
    return pl.pallas_call(
        _tc_b_body,
        out_shape=jax.ShapeDtypeStruct((N, width_out), jnp.float32),
    )(part, mat, selfw_col, b, w_next)


def _tc_b3_body(p_ref, mat_ref, selfw_ref, b_ref, w3_ref, out_ref):
    h = p_ref[0] + p_ref[1] + selfw_ref[...] * mat_ref[...] + b_ref[...]
    h = jnp.maximum(h, 0.0)
    out_ref[:, 0:O] = jnp.dot(h, w3_ref[...], preferred_element_type=jnp.float32)
    out_ref[:, O:H] = jnp.zeros((N, H - O), jnp.float32)


def _tc_b3(part, mat, selfw_col, b, w3):
    return pl.pallas_call(
        _tc_b3_body,
        out_shape=jax.ShapeDtypeStruct((N, H), jnp.float32),
    )(part, mat, selfw_col, b, w3)


def _tc_c_body(p_ref, mat3_ref, selfw_ref, b3_ref,
               wv1_ref, bv1_ref, wv2_ref, bv2_ref,
               wa1_ref, ba1_ref, wa2_ref, ba2_ref, q_ref):
    h = (p_ref[0][:, 0:O] + p_ref[1][:, 0:O]
         + selfw_ref[...] * mat3_ref[:, 0:O] + b3_ref[...])
    h = jnp.maximum(h, 0.0)
    v1 = jnp.maximum(jnp.dot(h, wv1_ref[...], preferred_element_type=jnp.float32)
                     + bv1_ref[...], 0.0)
    value = jnp.dot(v1, wv2_ref[...], preferred_element_type=jnp.float32) + bv2_ref[...]
    a1 = jnp.maximum(jnp.dot(h, wa1_ref[...], preferred_element_type=jnp.float32)
                     + ba1_ref[...], 0.0)
    adv = jnp.dot(a1, wa2_ref[...], preferred_element_type=jnp.float32) + ba2_ref[...]
    q_ref[...] = value + adv - jnp.sum(adv) / N


def _tc_c(part, mat3p, selfw_col, b3, wv1, bv1, wv2, bv2, wa1, ba1, wa2, ba2):
    return pl.pallas_call(
        _tc_c_body,
        out_shape=jax.ShapeDtypeStruct((N, 1), jnp.float32),
        compiler_params=pltpu.CompilerParams(vmem_limit_bytes=64 * 1024 * 1024),
    )(part, mat3p, selfw_col, b3.reshape(1, O),
      wv1, bv1.reshape(1, 32), wv2, bv2.reshape(1, 1),
      wa1, ba1.reshape(1, 32), wa2, ba2.reshape(1, 1))


def kernel(x, edge_index, edge_weight, W1, b1, W2, b2, W3, b3,
           Wv1, bv1, Wv2, bv2, Wa1, ba1, Wa2, ba2):
    src = edge_index[0]
    dst = edge_index[1]

    dst3 = dst.reshape(NW, NCH, CH)
    ew3 = edge_weight.reshape(NW, NCH, CH)
    deg_part = _deg_kernel(dst3, ew3).reshape(NC, N)
    mat1, dis, selfw = _tc_a(x, W1, deg_part)
    dis_flat = dis.reshape(N)
    selfw_col = selfw.reshape(N, 1)

    norm = _norm_kernel(src, dst, edge_weight, dis_flat)
    packed = jnp.stack(
        [src.reshape(NW, NCH, CH), dst.reshape(NW, NCH, CH),
         jax.lax.bitcast_convert_type(norm, jnp.int32).reshape(NW, NCH, CH)],
        axis=2).reshape(NW * NCH, 3, CH)

    part1 = _scatter_h(mat1, packed)
    mat2 = _tc_b(part1, mat1, selfw_col, b1.reshape(1, H), W2, H)

    part2 = _scatter_h(mat2, packed)
    mat3p = _tc_b3(part2, mat2, selfw_col, b2.reshape(1, H), W3)

    part3 = _scatter_h(mat3p, packed)
    q = _tc_c(part3, mat3p, selfw_col, b3, Wv1, bv1, Wv2, bv2, Wa1, ba1, Wa2, ba2)
    return q.reshape(N)


# speedup vs baseline: 1.0343x; 1.0343x over previous
import functools

import jax
import jax.numpy as jnp
from jax import lax
from jax.experimental import pallas as pl
from jax.experimental.pallas import tpu as pltpu
from jax.experimental.pallas import tpu_sc as plsc

N = 10000
E = 320000
D = 128
H = 128
O = 64

NC = 2
NS = 16
NW = NC * NS
ET = E // NW
CH = 80
NCH = ET // CH
RPT = 624
ZR = 16

_mesh = plsc.VectorSubcoreMesh(core_axis_name="c", subcore_axis_name="s")
_sc_params = pltpu.CompilerParams(needs_layout_passes=False)


def _worker_id():
    return lax.axis_index("s") * NC + lax.axis_index("c")


def _deg_body(dst_hbm, w_hbm, deg_hbm, dstv, wv, zbuf, sacc, ssem):
    cid = lax.axis_index("c")
    sid = lax.axis_index("s")
    wid = _worker_id()
    pltpu.sync_copy(dst_hbm.at[wid], dstv)
    pltpu.sync_copy(w_hbm.at[wid], wv)

    @pl.when(sid == 0)
    def _():
        def zb(i, _):
            zbuf[pl.ds(i * 16, 16)] = jnp.zeros((16,), jnp.float32)
            return 0
        lax.fori_loop(0, 2000 // 16, zb, 0)

        def zc(k, _):
            pltpu.sync_copy(zbuf, sacc.at[pl.ds(k * 2000, 2000)])
            return 0
        lax.fori_loop(0, N // 2000, zc, 0)

    plsc.subcore_barrier()

    def chunk(j, _):
        pltpu.sync_copy(wv.at[j], sacc.at[dstv.at[j]], add=True)
        return 0
    lax.fori_loop(0, NCH, chunk, 0)

    plsc.subcore_barrier()

    @pl.when(sid == 0)
    def _():
        def ro(k, _):
            pltpu.sync_copy(sacc.at[pl.ds(k * 2000, 2000)], zbuf)
            off = pl.multiple_of(cid * N + k * 2000, 8)
            pltpu.sync_copy(zbuf, deg_hbm.at[pl.ds(off, 2000)])
            return 0
        lax.fori_loop(0, N // 2000, ro, 0)


_deg_kernel = functools.partial(
    pl.kernel,
    out_type=jax.ShapeDtypeStruct((NC * N,), jnp.float32),
    mesh=_mesh,
    compiler_params=_sc_params,
    scratch_types=[
        pltpu.VMEM((NCH, CH), jnp.int32),
        pltpu.VMEM((NCH, CH), jnp.float32),
        pltpu.VMEM((2000,), jnp.float32),
        pltpu.VMEM_SHARED((N,), jnp.float32),
        pltpu.SemaphoreType.DMA,
    ],
)(_deg_body)


def _norm_body(src_hbm, dst_hbm, w_hbm, dis_hbm, norm_hbm,
               sv, dv, wv, disv, normv):
    wid = _worker_id()
    base = pl.multiple_of(wid * ET, 8)
    pltpu.sync_copy(src_hbm.at[pl.ds(base, ET)], sv)
    pltpu.sync_copy(dst_hbm.at[pl.ds(base, ET)], dv)
    pltpu.sync_copy(w_hbm.at[pl.ds(base, ET)], wv)
    pltpu.sync_copy(dis_hbm, disv)

    def grp(t, _):
        s16 = sv[pl.ds(t * 16, 16)]
        d16 = dv[pl.ds(t * 16, 16)]
        w16 = wv[pl.ds(t * 16, 16)]
        n16 = plsc.load_gather(disv, [s16]) * w16 * plsc.load_gather(disv, [d16])
        normv[pl.ds(t * 16, 16)] = n16
        return 0
    lax.fori_loop(0, ET // 16, grp, 0)

    pltpu.sync_copy(normv, norm_hbm.at[pl.ds(base, ET)])


_norm_kernel = functools.partial(
    pl.kernel,
    out_type=jax.ShapeDtypeStruct((E,), jnp.float32),
    mesh=_mesh,
    compiler_params=_sc_params,
    scratch_types=[
        pltpu.VMEM((ET,), jnp.int32),
        pltpu.VMEM((ET,), jnp.int32),
        pltpu.VMEM((ET,), jnp.float32),
        pltpu.VMEM((N,), jnp.float32),
        pltpu.VMEM((ET,), jnp.float32),
    ],
)(_norm_body)


def _scatter_body(width, mat_hbm, src_hbm, dst_hbm, norm_hbm, out_hbm,
                  sb0, db0, nb0, sb1, db1, nb1, rb0, rb1, rb2, sacc,
                  gsem0, gsem1, ssem, esem0, esem1):
    cid = lax.axis_index("c")
    sid = lax.axis_index("s")
    wid = _worker_id()

    def zrows(i, _):
        for c in range(width // 16):
            rb2[i, pl.ds(c * 16, 16)] = jnp.zeros((16,), jnp.float32)
        return 0
    lax.fori_loop(0, ZR, zrows, 0)

    row0 = sid * RPT
    nz = jnp.where(sid == NS - 1, (N - (NS - 1) * RPT) // ZR, RPT // ZR)

    def zc(k, _):
        r0 = pl.multiple_of(row0 + k * ZR, 8)
        pltpu.sync_copy(rb2.at[pl.ds(0, ZR)], sacc.at[pl.ds(r0, ZR)])
        return 0
    lax.fori_loop(0, nz, zc, 0)

    plsc.subcore_barrier()

    lane_consts = [jnp.full((16,), r, jnp.int32) for r in range(16)]
    ebufs = ((sb0, db0, nb0, esem0), (sb1, db1, nb1, esem1))
    rbufs = ((rb0, gsem0), (rb1, gsem1))

    def efetch(j, b):
        base = pl.multiple_of(wid * ET + j * CH, 8)
        sb, db, nb, esem = ebufs[b]
        pltpu.async_copy(src_hbm.at[pl.ds(base, CH)], sb, esem)
        pltpu.async_copy(dst_hbm.at[pl.ds(base, CH)], db, esem)
        pltpu.async_copy(norm_hbm.at[pl.ds(base, CH)], nb, esem)

    def edrain(b):
        sb, db, nb, esem = ebufs[b]
        pltpu.make_async_copy(src_hbm.at[pl.ds(0, CH)], sb, esem).wait()
        pltpu.make_async_copy(dst_hbm.at[pl.ds(0, CH)], db, esem).wait()
        pltpu.make_async_copy(norm_hbm.at[pl.ds(0, CH)], nb, esem).wait()

    def gather(b):
        sb = ebufs[b][0]
        rb, gsem = rbufs[b]
        pltpu.async_copy(mat_hbm.at[sb], rb, gsem)

    def gwait(b):
        rb, gsem = rbufs[b]
        pltpu.make_async_copy(mat_hbm.at[pl.ds(0, CH)], rb, gsem).wait()

    def swait():
        pltpu.make_async_copy(mat_hbm.at[pl.ds(0, CH)], rb2, ssem).wait()

    def scale(b):
        nb = ebufs[b][2]
        rb = rbufs[b][0]

        def grp(g, _):
            n16 = nb[pl.ds(g * 16, 16)]
            for r0 in range(0, 16, 2):
                rows = [g * 16 + r0, g * 16 + r0 + 1]
                bcs = [jnp.take(n16, lane_consts[r0 + i]) for i in range(2)]
                vals = [[rb[rows[i], pl.ds(c * 16, 16)] for c in range(width // 16)]
                        for i in range(2)]
                for i in range(2):
                    for c in range(width // 16):
                        rb2[rows[i], pl.ds(c * 16, 16)] = vals[i][c] * bcs[i]
            return 0
        lax.fori_loop(0, CH // 16, grp, 0)

    def scat(b):
        db = ebufs[b][1]
        pltpu.async_copy(rb2, sacc.at[db], ssem, add=True)

    efetch(0, 0)
    edrain(0)
    gather(0)

    def pair(k, _):
        @pl.when(k > 0)
        def _():
            swait()
        efetch(2 * k + 1, 1)
        edrain(1)
        gather(1)
        gwait(0)
        scale(0)
        scat(0)

        swait()
        efetch(2 * k + 2, 0)
        edrain(0)
        gather(0)
        gwait(1)
        scale(1)
        scat(1)
        return 0
    lax.fori_loop(0, (NCH - 1) // 2, pair, 0)

    swait()
    gwait(0)
    scale(0)
    pltpu.sync_copy(rb2, sacc.at[db0], add=True)

    plsc.subcore_barrier()

    def ro(k, _):
        r0 = pl.multiple_of(row0 + k * ZR, 8)
        pltpu.sync_copy(sacc.at[pl.ds(r0, ZR)], rb0.at[pl.ds(0, ZR)])
        pltpu.sync_copy(rb0.at[pl.ds(0, ZR)], out_hbm.at[cid, pl.ds(r0, ZR)])
        return 0
    lax.fori_loop(0, nz, ro, 0)


def _make_scatter_kernel(width):
    return functools.partial(
        pl.kernel,
        out_type=jax.ShapeDtypeStruct((NC, N, width), jnp.float32),
        mesh=_mesh,
        compiler_params=_sc_params,
        scratch_types=[
            pltpu.VMEM((CH,), jnp.int32),
            pltpu.VMEM((CH,), jnp.int32),
            pltpu.VMEM((CH,), jnp.float32),
            pltpu.VMEM((CH,), jnp.int32),
            pltpu.VMEM((CH,), jnp.int32),
            pltpu.VMEM((CH,), jnp.float32),
            pltpu.VMEM((CH, width), jnp.float32),
            pltpu.VMEM((CH, width), jnp.float32),
            pltpu.VMEM((CH, width), jnp.float32),
            pltpu.VMEM_SHARED((N, width), jnp.float32),
            pltpu.SemaphoreType.DMA,
            pltpu.SemaphoreType.DMA,
            pltpu.SemaphoreType.DMA,
            pltpu.SemaphoreType.DMA,
            pltpu.SemaphoreType.DMA,
        ],
    )(functools.partial(_scatter_body, width))


_scatter_h = _make_scatter_kernel(H)


def _tc_a_body(x_ref, w_ref, deg_ref, mat_ref, dis_ref, selfw_ref):
    deg = deg_ref[0:1, :] + deg_ref[1:2, :] + 1.0
    dis_ref[...] = lax.rsqrt(deg)
    selfw_ref[...] = 1.0 / deg
    mat_ref[...] = jnp.dot(x_ref[...], w_ref[...],
                           preferred_element_type=jnp.float32)


def _tc_a(x, w1, deg_part):
    return pl.pallas_call(
        _tc_a_body,
        out_shape=[
            jax.ShapeDtypeStruct((N, H), jnp.float32),
            jax.ShapeDtypeStruct((1, N), jnp.float32),
            jax.ShapeDtypeStruct((1, N), jnp.float32),
        ],
    )(x, w1, deg_part)


def _tc_b_body(p_ref, mat_ref, selfw_ref, b_ref, wn_ref, out_ref):
    h = p_ref[0] + p_ref[1] + selfw_ref[...] * mat_ref[...] + b_ref[...]
    h = jnp.maximum(h, 0.0)
    out_ref[...] = jnp.dot(h, wn_ref[...], preferred_element_type=jnp.float32)


def _tc_b(part, mat, selfw_col, b, w_next, width_out):
    return pl.pallas_call(
        _tc_b_body,
        out_shape=jax.ShapeDtypeStruct((N, width_out), jnp.float32),
    )(part, mat, selfw_col, b, w_next)


def _tc_b3_body(p_ref, mat_ref, selfw_ref, b_ref, w3_ref, out_ref):
    h = p_ref[0] + p_ref[1] + selfw_ref[...] * mat_ref[...] + b_ref[...]
    h = jnp.maximum(h, 0.0)
    out_ref[:, 0:O] = jnp.dot(h, w3_ref[...], preferred_element_type=jnp.float32)
    out_ref[:, O:H] = jnp.zeros((N, H - O), jnp.float32)


def _tc_b3(part, mat, selfw_col, b, w3):
    return pl.pallas_call(
        _tc_b3_body,
        out_shape=jax.ShapeDtypeStruct((N, H), jnp.float32),
    )(part, mat, selfw_col, b, w3)


def _tc_c_body(p_ref, mat3_ref, selfw_ref, b3_ref,
               wv1_ref, bv1_ref, wv2_ref, bv2_ref,
               wa1_ref, ba1_ref, wa2_ref, ba2_ref, q_ref):
    h = (p_ref[0][:, 0:O] + p_ref[1][:, 0:O]
         + selfw_ref[...] * mat3_ref[:, 0:O] + b3_ref[...])
    h = jnp.maximum(h, 0.0)
    v1 = jnp.maximum(jnp.dot(h, wv1_ref[...], preferred_element_type=jnp.float32)
                     + bv1_ref[...], 0.0)
    value = jnp.dot(v1, wv2_ref[...], preferred_element_type=jnp.float32) + bv2_ref[...]
    a1 = jnp.maximum(jnp.dot(h, wa1_ref[...], preferred_element_type=jnp.float32)
                     + ba1_ref[...], 0.0)
    adv = jnp.dot(a1, wa2_ref[...], preferred_element_type=jnp.float32) + ba2_ref[...]
    q_ref[...] = value + adv - jnp.sum(adv) / N


def _tc_c(part, mat3p, selfw_col, b3, wv1, bv1, wv2, bv2, wa1, ba1, wa2, ba2):
    return pl.pallas_call(
        _tc_c_body,
        out_shape=jax.ShapeDtypeStruct((N, 1), jnp.float32),
        compiler_params=pltpu.CompilerParams(vmem_limit_bytes=64 * 1024 * 1024),
    )(part, mat3p, selfw_col, b3.reshape(1, O),
      wv1, bv1.reshape(1, 32), wv2, bv2.reshape(1, 1),
      wa1, ba1.reshape(1, 32), wa2, ba2.reshape(1, 1))


def kernel(x, edge_index, edge_weight, W1, b1, W2, b2, W3, b3,
           Wv1, bv1, Wv2, bv2, Wa1, ba1, Wa2, ba2):
    src = edge_index[0]
    dst = edge_index[1]

    dst3 = dst.reshape(NW, NCH, CH)
    ew3 = edge_weight.reshape(NW, NCH, CH)
    deg_part = _deg_kernel(dst3, ew3).reshape(NC, N)
    mat1, dis, selfw = _tc_a(x, W1, deg_part)
    dis_flat = dis.reshape(N)
    selfw_col = selfw.reshape(N, 1)

    norm = _norm_kernel(src, dst, edge_weight, dis_flat)

    part1 = _scatter_h(mat1, src, dst, norm)
    mat2 = _tc_b(part1, mat1, selfw_col, b1.reshape(1, H), W2, H)

    part2 = _scatter_h(mat2, src, dst, norm)
    mat3p = _tc_b3(part2, mat2, selfw_col, b2.reshape(1, H), W3)

    part3 = _scatter_h(mat3p, src, dst, norm)
    q = _tc_c(part3, mat3p, selfw_col, b3, Wv1, bv1, Wv2, bv2, Wa1, ba1, Wa2, ba2)
    return q.reshape(N)
